# Initial kernel scaffold; baseline (speedup 1.0000x reference)
import jax
import jax.numpy as jnp
from jax.experimental import pallas as pl


def kernel(start, query, kb_triple, kb_range, rel_emb, step_W, step_b, cls_W, cls_b, W_ih, W_hh, b_ih, b_hh):
    raise NotImplementedError("write your pallas kernel here")


# speedup vs baseline: 85.2296x; 85.2296x over previous
import functools
import jax
import jax.numpy as jnp
import numpy as np
from jax import lax
from jax.experimental import pallas as pl
from jax.experimental.pallas import tpu as pltpu
from jax.experimental.pallas import tpu_sc as plsc

DIM = 128
NUM_ENT = 10000
NUM_REL = 200
N_TRIPLES = 160000
BSZ = 16
MAX_ACTIVE = 400
THRES = 0.7
L = 16
ENT_CHUNKS = NUM_ENT // L
SLOT_CHUNKS = MAX_ACTIVE // L
CAP = 512
NEG = np.float32(-3.0e38)

def _lanes():
  return lax.broadcasted_iota(jnp.int32, (L,), 0)


def _select_and_gather(src_v, r0_v, r1_v, kbf_hbm, act_v, c_v, s0_v, nb_v,
                       tri_v, sem,
                       sub_v, obj_v, rel_v, lesub_v, epsub_v, valid_v,
                       with_ep):
  lane = _lanes()
  zero_chunk = jnp.zeros((L,), jnp.int32)

  def zero_act(c, _):
    act_v[pl.ds(c * L, L)] = zero_chunk
    return 0

  lax.fori_loop(0, CAP // L, zero_act, 0)

  def scan_body(c, carry):
    na, best, bestidx = carry
    v = src_v[pl.ds(c * L, L)]
    e = c * L + lane
    not0 = e != 0
    m = (v > THRES) & not0
    mi = m.astype(jnp.int32)
    cnt = jnp.sum(mi)
    pos = na + plsc.cumsum(mi) - 1
    plsc.store_scatter(act_v, [pos], e, mask=m)
    vm = jnp.where(not0, v, NEG)
    cmax = jnp.max(vm)
    carg = jnp.min(jnp.where((vm == cmax) & not0, e, np.int32(2 ** 30)))
    better = cmax > best
    best = jnp.where(better, cmax, best)
    bestidx = jnp.where(better, carg, bestidx)
    return na + cnt, best, bestidx

  na, _, bestidx = lax.fori_loop(0, ENT_CHUNKS, scan_body,
                                 (np.int32(0), NEG, np.int32(1)))

  @pl.when(na == 0)
  def _():
    act_v[pl.ds(0, L)] = jnp.full((L,), bestidx, jnp.int32)

  k = jnp.maximum(na, 1)
  nk = lax.div(k + (L - 1), L)

  def rng_body(j, tot):
    gi = j * L + lane
    lm = gi < k
    e = act_v[pl.ds(j * L, L)]
    r0 = plsc.load_gather(r0_v, [e])
    r1 = plsc.load_gather(r1_v, [e])
    ln = jnp.where(lm, r1 - r0, 0)
    inc = plsc.cumsum(ln)
    cex = tot + inc - ln
    room = jnp.maximum(MAX_ACTIVE - cex, 0)
    cl = jnp.minimum(ln, room)
    c_v[pl.ds(j * L, L)] = cex
    s0_v[pl.ds(j * L, L)] = r0
    nb_v[pl.ds(j * L, L)] = lax.div(cl + (L - 1), L)
    return tot + jnp.max(inc)

  total = lax.fori_loop(0, nk, rng_body, np.int32(0))
  count = jnp.minimum(total, MAX_ACTIVE)

  def ent_body(j, nd):
    nb = nb_v[pl.ds(j, L)][0]
    c0 = c_v[pl.ds(j, L)][0]
    s0 = s0_v[pl.ds(j, L)][0]

    def blk(q, nd2):
      pltpu.async_copy(kbf_hbm.at[pl.ds((s0 + q * L) * 8, 128)],
                       tri_v.at[pl.ds((c0 + q * L) * 8, 128)], sem)
      return nd2 + 1

    return lax.fori_loop(0, nb, blk, nd)

  ndma = lax.fori_loop(0, k, ent_body, np.int32(0))

  def drain(_, x):
    pltpu.make_async_copy(kbf_hbm.at[pl.ds(0, 128)],
                          tri_v.at[pl.ds(0, 128)], sem).wait()
    return x

  lax.fori_loop(0, ndma, drain, 0)

  def col_body(s, _):
    p = (s * L + lane) * 8
    sub = jnp.clip(plsc.load_gather(tri_v, [p]), 0, NUM_ENT - 1)
    obj = jnp.clip(plsc.load_gather(tri_v, [p + 1]), 0, NUM_ENT - 1)
    rel = jnp.clip(plsc.load_gather(tri_v, [p + 2]), 0, NUM_REL - 1)
    sub_v[pl.ds(s * L, L)] = sub
    obj_v[pl.ds(s * L, L)] = obj
    rel_v[pl.ds(s * L, L)] = rel
    sv = plsc.load_gather(src_v, [sub])
    if with_ep:
      epsub_v[pl.ds(s * L, L)] = sv
      lesub_v[pl.ds(s * L, L)] = jnp.minimum(sv, 1.0)
    else:
      lesub_v[pl.ds(s * L, L)] = sv
    valid_v[pl.ds(s * L, L)] = ((s * L + lane) < count).astype(jnp.float32)
    return 0

  lax.fori_loop(0, SLOT_CHUNKS, col_body, 0)


def _scatter_ep(ep_v, objrow_v, objprow_v):
  zf = jnp.zeros((L,), jnp.float32)

  def zb(c, _):
    ep_v[pl.ds(c * L, L)] = zf
    return 0

  lax.fori_loop(0, ENT_CHUNKS, zb, 0)

  def sc_body(s, _):
    o = objrow_v[pl.ds(s * L, L)]
    p = objprow_v[pl.ds(s * L, L)]
    plsc.addupdate_scatter(ep_v, [o], p)
    return 0

  lax.fori_loop(0, SLOT_CHUNKS, sc_body, 0)


@functools.lru_cache(maxsize=1)
def _build_sc_kernels():
  mesh = plsc.VectorSubcoreMesh(core_axis_name="c", subcore_axis_name="s")
  cp = pltpu.CompilerParams(needs_layout_passes=False)
  sel_scratch = [
      pltpu.VMEM((NUM_ENT,), jnp.float32),
      pltpu.VMEM((NUM_ENT,), jnp.int32),
      pltpu.VMEM((NUM_ENT,), jnp.int32),
      pltpu.VMEM((CAP,), jnp.int32),
      pltpu.VMEM((CAP,), jnp.int32),
      pltpu.VMEM((CAP,), jnp.int32),
      pltpu.VMEM((CAP,), jnp.int32),
      pltpu.VMEM((CAP * 8,), jnp.int32),
      pltpu.VMEM((MAX_ACTIVE,), jnp.int32),
      pltpu.VMEM((MAX_ACTIVE,), jnp.int32),
      pltpu.VMEM((MAX_ACTIVE,), jnp.int32),
      pltpu.VMEM((MAX_ACTIVE,), jnp.float32),
      pltpu.VMEM((MAX_ACTIVE,), jnp.float32),
      pltpu.VMEM((MAX_ACTIVE,), jnp.float32),
      pltpu.VMEM((MAX_ACTIVE,), jnp.float32),
      pltpu.SemaphoreType.DMA,
  ]
  rows_i = jax.ShapeDtypeStruct((BSZ, MAX_ACTIVE), jnp.int32)
  rows_f = jax.ShapeDtypeStruct((BSZ, MAX_ACTIVE), jnp.float32)

  def out_rows(i, pairs):
    for vec, hbm in pairs:
      pltpu.sync_copy(vec, hbm.at[i])

  @functools.partial(
      pl.kernel, mesh=mesh,
      out_type=(rows_i, rows_i, rows_i, rows_f, rows_f),
      scratch_types=sel_scratch, compiler_params=cp)
  def sc1(start_hbm, r0_hbm, r1_hbm, kbf_hbm,
          sub_o, obj_o, rel_o, lesub_o, valid_o,
          src_v, r0_v, r1_v, act_v, c_v, s0_v, nb_v, tri_v,
          sub_v, obj_v, rel_v, lesub_v, epsub_v, objprow_v, valid_v, sem):
    wid = lax.axis_index("s") * 2 + lax.axis_index("c")

    @pl.when(wid < BSZ)
    def _():
      cp0 = pltpu.async_copy(r0_hbm, r0_v, sem)
      cp1 = pltpu.async_copy(r1_hbm, r1_v, sem)
      pltpu.sync_copy(start_hbm.at[wid], src_v)
      cp0.wait()
      cp1.wait()
      _select_and_gather(src_v, r0_v, r1_v, kbf_hbm, act_v, c_v, s0_v, nb_v,
                         tri_v, sem,
                         sub_v, obj_v, rel_v, lesub_v, epsub_v, valid_v,
                         with_ep=False)
      out_rows(wid, [(sub_v, sub_o), (obj_v, obj_o), (rel_v, rel_o),
                     (lesub_v, lesub_o), (valid_v, valid_o)])

  @functools.partial(
      pl.kernel, mesh=mesh,
      out_type=(rows_i, rows_i, rows_i, rows_f, rows_f, rows_f),
      scratch_types=sel_scratch + [pltpu.VMEM((MAX_ACTIVE,), jnp.int32)],
      compiler_params=cp)
  def sc2(obj_hbm, objp_hbm, r0_hbm, r1_hbm, kbf_hbm,
          sub_o, obj_o, rel_o, lesub_o, epsub_o, valid_o,
          src_v, r0_v, r1_v, act_v, c_v, s0_v, nb_v, tri_v,
          sub_v, obj_v, rel_v, lesub_v, epsub_v, objprow_v, valid_v, sem,
          pobj_v):
    wid = lax.axis_index("s") * 2 + lax.axis_index("c")

    @pl.when(wid < BSZ)
    def _():
      cp0 = pltpu.async_copy(r0_hbm, r0_v, sem)
      cp1 = pltpu.async_copy(r1_hbm, r1_v, sem)
      pltpu.sync_copy(obj_hbm.at[wid], pobj_v)
      pltpu.sync_copy(objp_hbm.at[wid], objprow_v)
      _scatter_ep(src_v, pobj_v, objprow_v)
      cp0.wait()
      cp1.wait()
      _select_and_gather(src_v, r0_v, r1_v, kbf_hbm, act_v, c_v, s0_v, nb_v,
                         tri_v, sem,
                         sub_v, obj_v, rel_v, lesub_v, epsub_v, valid_v,
                         with_ep=True)
      out_rows(wid, [(sub_v, sub_o), (obj_v, obj_o), (rel_v, rel_o),
                     (lesub_v, lesub_o), (epsub_v, epsub_o),
                     (valid_v, valid_o)])

  @functools.partial(
      pl.kernel, mesh=mesh,
      out_type=jax.ShapeDtypeStruct((BSZ, NUM_ENT), jnp.float32),
      scratch_types=[
          pltpu.VMEM((NUM_ENT,), jnp.float32),
          pltpu.VMEM((MAX_ACTIVE,), jnp.int32),
          pltpu.VMEM((MAX_ACTIVE,), jnp.float32),
      ], compiler_params=cp)
  def sc3(obj_hbm, objp_hbm, out_hbm, ep_v, objrow_v, objprow_v):
    wid = lax.axis_index("s") * 2 + lax.axis_index("c")

    @pl.when(wid < BSZ)
    def _():
      pltpu.sync_copy(obj_hbm.at[wid], objrow_v)
      pltpu.sync_copy(objp_hbm.at[wid], objprow_v)
      _scatter_ep(ep_v, objrow_v, objprow_v)

      def norm_body(c, _):
        ep_v[pl.ds(c * L, L)] = jnp.minimum(ep_v[pl.ds(c * L, L)], 1.0)
        return 0

      lax.fori_loop(0, ENT_CHUNKS, norm_body, 0)
      pltpu.sync_copy(ep_v, out_hbm.at[wid])

  return sc1, sc2, sc3


def _gates(gi, gh):
  i_r, i_z, i_n = gi[:, :DIM], gi[:, DIM:2 * DIM], gi[:, 2 * DIM:]
  h_r, h_z, h_n = gh[:, :DIM], gh[:, DIM:2 * DIM], gh[:, 2 * DIM:]
  r = jax.nn.sigmoid(i_r + h_r)
  z = jax.nn.sigmoid(i_z + h_z)
  n = jnp.tanh(i_n + r * h_n)
  return z, n


def _tc1_body(rel_ref, lesub_ref, valid_ref, q1h_ref,
              rel_emb_ref, W_ih_ref, b_ih_ref, b_hh_ref,
              sw_ref, sb_ref, clsw_ref, clsb_ref,
              objp_ref, feat_ref):
  rel = rel_ref[0, 0]
  oh = (rel[:, None] ==
        lax.broadcasted_iota(jnp.int32, (MAX_ACTIVE, NUM_REL), 1)
        ).astype(jnp.float32)
  rel_feat = jnp.dot(oh, rel_emb_ref[...],
                     preferred_element_type=jnp.float32)
  gi = jnp.dot(rel_feat, W_ih_ref[...],
               preferred_element_type=jnp.float32) + b_ih_ref[...]
  gh = jnp.broadcast_to(b_hh_ref[...], (MAX_ACTIVE, 3 * DIM))
  z, n = _gates(gi, gh)
  trans = (1.0 - z) * n
  qe = jnp.dot(q1h_ref[0], rel_emb_ref[...],
               preferred_element_type=jnp.float32)
  cq = jnp.tanh(jnp.dot(qe, sw_ref[...],
                        preferred_element_type=jnp.float32) + sb_ref[...])
  logit = jnp.sum(trans * cq * clsw_ref[...], axis=1) + clsb_ref[0, 0]
  prob = jax.nn.sigmoid(logit)
  obj_p = lesub_ref[0, 0] * prob * valid_ref[0, 0]
  objp_ref[0, 0] = obj_p
  feat_ref[0] = trans * obj_p[:, None]


def _tc2_body(rel_ref, sub_ref, lesub_ref, epsub_ref, valid_ref,
              pobj_ref, pfeat_ref, q1h_ref,
              rel_emb_ref, W_ih_ref, W_hh_ref, b_ih_ref, b_hh_ref,
              sw_ref, sb_ref, clsw_ref, clsb_ref,
              objp_ref):
  rel = rel_ref[0, 0]
  oh = (rel[:, None] ==
        lax.broadcasted_iota(jnp.int32, (MAX_ACTIVE, NUM_REL), 1)
        ).astype(jnp.float32)
  rel_feat = jnp.dot(oh, rel_emb_ref[...],
                     preferred_element_type=jnp.float32)
  gi = jnp.dot(rel_feat, W_ih_ref[...],
               preferred_element_type=jnp.float32) + b_ih_ref[...]
  mask = (sub_ref[0, 0][:, None] == pobj_ref[0, 0][None, :]
          ).astype(jnp.float32)
  S = jnp.dot(mask, pfeat_ref[0], preferred_element_type=jnp.float32)
  h = S / (epsub_ref[0, 0][:, None] + 1e-6)
  gh = jnp.dot(h, W_hh_ref[...],
               preferred_element_type=jnp.float32) + b_hh_ref[...]
  z, n = _gates(gi, gh)
  trans = (1.0 - z) * n + z * h
  qe = jnp.dot(q1h_ref[0], rel_emb_ref[...],
               preferred_element_type=jnp.float32)
  cq = jnp.tanh(jnp.dot(qe, sw_ref[...],
                        preferred_element_type=jnp.float32) + sb_ref[...])
  logit = jnp.sum(trans * cq * clsw_ref[...], axis=1) + clsb_ref[0, 0]
  prob = jax.nn.sigmoid(logit)
  objp_ref[0, 0] = lesub_ref[0, 0] * prob * valid_ref[0, 0]


def _i3(x):
  return x.reshape(BSZ, 1, MAX_ACTIVE)


_B3 = lambda: pl.BlockSpec((1, 1, MAX_ACTIVE), lambda i: (i, 0, 0))
_BQ = lambda: pl.BlockSpec((1, 1, NUM_REL), lambda i: (i, 0, 0))
_BW = lambda shape: pl.BlockSpec(shape, lambda i: tuple(0 for _ in shape))


def kernel(start, query, kb_triple, kb_range, rel_emb, step_W, step_b,
           cls_W, cls_b, W_ih, W_hh, b_ih, b_hh):
  f32 = jnp.float32
  kbf = jnp.pad(kb_triple.astype(jnp.int32), ((0, 16), (0, 5))).reshape(-1)
  r0 = kb_range[:, 0].astype(jnp.int32)
  r1 = kb_range[:, 1].astype(jnp.int32)
  q1h = (query[:, None].astype(jnp.int32) ==
         jnp.arange(NUM_REL, dtype=jnp.int32)[None, :]).astype(f32)
  q1h = q1h.reshape(BSZ, 1, NUM_REL)
  b_ih2 = b_ih.reshape(1, 3 * DIM).astype(f32)
  b_hh2 = b_hh.reshape(1, 3 * DIM).astype(f32)
  clsw = cls_W.reshape(1, DIM).astype(f32)
  clsb = cls_b.reshape(1, 1).astype(f32)
  sb = step_b.reshape(2, 1, DIM).astype(f32)
  _sc1, _sc2, _sc3 = _build_sc_kernels()

  sub1, obj1, rel1, lesub1, valid1 = _sc1(start.astype(f32), r0, r1, kbf)

  tc1 = pl.pallas_call(
      _tc1_body,
      grid=(BSZ,),
      in_specs=[_B3(), _B3(), _B3(), _BQ(),
                _BW((NUM_REL, DIM)), _BW((DIM, 3 * DIM)), _BW((1, 3 * DIM)),
                _BW((1, 3 * DIM)), _BW((DIM, DIM)), _BW((1, DIM)),
                _BW((1, DIM)), _BW((1, 1))],
      out_specs=[_B3(), pl.BlockSpec((1, MAX_ACTIVE, DIM), lambda i: (i, 0, 0))],
      out_shape=[jax.ShapeDtypeStruct((BSZ, 1, MAX_ACTIVE), f32),
                 jax.ShapeDtypeStruct((BSZ, MAX_ACTIVE, DIM), f32)],
  )
  objp1, feat1 = tc1(_i3(rel1), _i3(lesub1), _i3(valid1), q1h,
                     rel_emb.astype(f32), W_ih.astype(f32).T, b_ih2, b_hh2,
                     step_W[0].astype(f32), sb[0], clsw, clsb)

  sub2, obj2, rel2, lesub2, epsub2, valid2 = _sc2(
      obj1, objp1.reshape(BSZ, MAX_ACTIVE), r0, r1, kbf)

  tc2 = pl.pallas_call(
      _tc2_body,
      grid=(BSZ,),
      in_specs=[_B3(), _B3(), _B3(), _B3(), _B3(), _B3(),
                pl.BlockSpec((1, MAX_ACTIVE, DIM), lambda i: (i, 0, 0)), _BQ(),
                _BW((NUM_REL, DIM)), _BW((DIM, 3 * DIM)), _BW((DIM, 3 * DIM)),
                _BW((1, 3 * DIM)), _BW((1, 3 * DIM)),
                _BW((DIM, DIM)), _BW((1, DIM)), _BW((1, DIM)), _BW((1, 1))],
      out_specs=[_B3()],
      out_shape=[jax.ShapeDtypeStruct((BSZ, 1, MAX_ACTIVE), f32)],
  )
  (objp2,) = tc2(_i3(rel2), _i3(sub2), _i3(lesub2), _i3(epsub2), _i3(valid2),
                 _i3(obj1), feat1, q1h,
                 rel_emb.astype(f32), W_ih.astype(f32).T, W_hh.astype(f32).T,
                 b_ih2, b_hh2, step_W[1].astype(f32), sb[1], clsw, clsb)

  return _sc3(obj2, objp2.reshape(BSZ, MAX_ACTIVE))


# speedup vs baseline: 90.2008x; 1.0583x over previous
import functools
import jax
import jax.numpy as jnp
import numpy as np
from jax import lax
from jax.experimental import pallas as pl
from jax.experimental.pallas import tpu as pltpu
from jax.experimental.pallas import tpu_sc as plsc

DIM = 128
NUM_ENT = 10000
NUM_REL = 200
N_TRIPLES = 160000
BSZ = 16
MAX_ACTIVE = 400
THRES = 0.7
L = 16
ENT_CHUNKS = NUM_ENT // L
SLOT_CHUNKS = MAX_ACTIVE // L
CAP = 512
NEG = np.float32(-3.0e38)

def _lanes():
  return lax.broadcasted_iota(jnp.int32, (L,), 0)


def _select_and_gather(src_v, r0_v, r1_v, kbf_hbm, act_v, c_v, s0_v, nb_v,
                       tri_v, sem,
                       sub_v, obj_v, rel_v, lesub_v, epsub_v, valid_v,
                       with_ep):
  lane = _lanes()
  zero_chunk = jnp.zeros((L,), jnp.int32)

  def zero_act(c, _):
    for u in range(8):
      act_v[pl.ds((c * 8 + u) * L, L)] = zero_chunk
    return 0

  lax.fori_loop(0, CAP // L // 8, zero_act, 0)

  head = src_v[pl.ds(0, L)]
  src_v[pl.ds(0, L)] = jnp.where(lane == 0, -1.0, head)

  nb_v[pl.ds(0, L)] = zero_chunk

  GRP = 8
  NGRP = ENT_CHUNKS // GRP

  def compact_chunk(c, m):
    mi = m.astype(jnp.int32)
    na0 = nb_v[pl.ds(0, L)][0]
    cnt = plsc.all_reduce_population_count(m)
    pos = na0 + plsc.cumsum(mi) - 1
    e = c * L + lane
    plsc.store_scatter(act_v, [pos], e, mask=m)
    nb_v[pl.ds(0, L)] = na0 + cnt

  def group_body(g, _):
    base = g * GRP
    ms = [src_v[pl.ds((base + u) * L, L)] > THRES for u in range(GRP)]
    acc = ms[0]
    for u in range(1, GRP):
      acc = acc | ms[u]

    @pl.when(jnp.any(acc))
    def _():
      for u in range(GRP):
        @pl.when(jnp.any(ms[u]))
        def _(u=u):
          compact_chunk(base + u, ms[u])

    return 0

  lax.fori_loop(0, NGRP, group_body, 0)
  for c in range(NGRP * GRP, ENT_CHUNKS):
    mt = src_v[pl.ds(c * L, L)] > THRES

    @pl.when(jnp.any(mt))
    def _(c=c, mt=mt):
      compact_chunk(c, mt)

  na = nb_v[pl.ds(0, L)][0]

  @pl.when(na == 0)
  def _():
    def fb_body(c, carry):
      best, bestidx = carry
      v = src_v[pl.ds(c * L, L)]
      cmax = jnp.max(v)
      carg = jnp.min(jnp.where(v == cmax, c * L + lane, np.int32(2 ** 30)))
      better = cmax > best
      return (jnp.where(better, cmax, best),
              jnp.where(better, carg, bestidx))

    _, bestidx = lax.fori_loop(0, ENT_CHUNKS, fb_body, (NEG, np.int32(1)))
    act_v[pl.ds(0, L)] = jnp.full((L,), bestidx, jnp.int32)

  k = jnp.maximum(na, 1)
  nk = lax.div(k + (L - 1), L)

  def rng_body(j, tot):
    gi = j * L + lane
    lm = gi < k
    e = act_v[pl.ds(j * L, L)]
    r0 = plsc.load_gather(r0_v, [e])
    r1 = plsc.load_gather(r1_v, [e])
    ln = jnp.where(lm, r1 - r0, 0)
    inc = plsc.cumsum(ln)
    cex = tot + inc - ln
    room = jnp.maximum(MAX_ACTIVE - cex, 0)
    cl = jnp.minimum(ln, room)
    c_v[pl.ds(j * L, L)] = cex
    s0_v[pl.ds(j * L, L)] = r0
    nb_v[pl.ds(j * L, L)] = lax.div(cl + (L - 1), L)
    return tot + jnp.max(inc)

  total = lax.fori_loop(0, nk, rng_body, np.int32(0))
  count = jnp.minimum(total, MAX_ACTIVE)

  def ent_body(j, nd):
    nb = nb_v[pl.ds(j, L)][0]
    c0 = c_v[pl.ds(j, L)][0]
    s0 = s0_v[pl.ds(j, L)][0]

    def blk(q, nd2):
      pltpu.async_copy(kbf_hbm.at[pl.ds((s0 + q * L) * 8, 128)],
                       tri_v.at[pl.ds((c0 + q * L) * 8, 128)], sem)
      return nd2 + 1

    return lax.fori_loop(0, nb, blk, nd)

  ndma = lax.fori_loop(0, k, ent_body, np.int32(0))

  def drain(_, x):
    pltpu.make_async_copy(kbf_hbm.at[pl.ds(0, 128)],
                          tri_v.at[pl.ds(0, 128)], sem).wait()
    return x

  lax.fori_loop(0, ndma, drain, 0)

  def col_body(s, _):
    p = (s * L + lane) * 8
    sub = jnp.clip(plsc.load_gather(tri_v, [p]), 0, NUM_ENT - 1)
    obj = jnp.clip(plsc.load_gather(tri_v, [p + 1]), 0, NUM_ENT - 1)
    rel = jnp.clip(plsc.load_gather(tri_v, [p + 2]), 0, NUM_REL - 1)
    sub_v[pl.ds(s * L, L)] = sub
    obj_v[pl.ds(s * L, L)] = obj
    rel_v[pl.ds(s * L, L)] = rel
    sv = plsc.load_gather(src_v, [sub])
    if with_ep:
      epsub_v[pl.ds(s * L, L)] = sv
      lesub_v[pl.ds(s * L, L)] = jnp.minimum(sv, 1.0)
    else:
      lesub_v[pl.ds(s * L, L)] = sv
    valid_v[pl.ds(s * L, L)] = ((s * L + lane) < count).astype(jnp.float32)
    return 0

  lax.fori_loop(0, SLOT_CHUNKS, col_body, 0)


def _scatter_ep(ep_v, objrow_v, objprow_v):
  zf = jnp.zeros((L,), jnp.float32)

  def zb(c, _):
    for u in range(5):
      ep_v[pl.ds((c * 5 + u) * L, L)] = zf
    return 0

  lax.fori_loop(0, ENT_CHUNKS // 5, zb, 0)

  def sc_body(s, _):
    o = objrow_v[pl.ds(s * L, L)]
    p = objprow_v[pl.ds(s * L, L)]
    plsc.addupdate_scatter(ep_v, [o], p)
    return 0

  lax.fori_loop(0, SLOT_CHUNKS, sc_body, 0)


@functools.lru_cache(maxsize=1)
def _build_sc_kernels():
  mesh = plsc.VectorSubcoreMesh(core_axis_name="c", subcore_axis_name="s")
  cp = pltpu.CompilerParams(needs_layout_passes=False)
  sel_scratch = [
      pltpu.VMEM((NUM_ENT,), jnp.float32),
      pltpu.VMEM((NUM_ENT,), jnp.int32),
      pltpu.VMEM((NUM_ENT,), jnp.int32),
      pltpu.VMEM((CAP,), jnp.int32),
      pltpu.VMEM((CAP,), jnp.int32),
      pltpu.VMEM((CAP,), jnp.int32),
      pltpu.VMEM((CAP,), jnp.int32),
      pltpu.VMEM((CAP * 8,), jnp.int32),
      pltpu.VMEM((MAX_ACTIVE,), jnp.int32),
      pltpu.VMEM((MAX_ACTIVE,), jnp.int32),
      pltpu.VMEM((MAX_ACTIVE,), jnp.int32),
      pltpu.VMEM((MAX_ACTIVE,), jnp.float32),
      pltpu.VMEM((MAX_ACTIVE,), jnp.float32),
      pltpu.VMEM((MAX_ACTIVE,), jnp.float32),
      pltpu.VMEM((MAX_ACTIVE,), jnp.float32),
      pltpu.SemaphoreType.DMA,
  ]
  rows_i = jax.ShapeDtypeStruct((BSZ, MAX_ACTIVE), jnp.int32)
  rows_f = jax.ShapeDtypeStruct((BSZ, MAX_ACTIVE), jnp.float32)

  def out_rows(i, pairs):
    for vec, hbm in pairs:
      pltpu.sync_copy(vec, hbm.at[i])

  @functools.partial(
      pl.kernel, mesh=mesh,
      out_type=(rows_i, rows_i, rows_i, rows_f, rows_f),
      scratch_types=sel_scratch, compiler_params=cp)
  def sc1(start_hbm, r0_hbm, r1_hbm, kbf_hbm,
          sub_o, obj_o, rel_o, lesub_o, valid_o,
          src_v, r0_v, r1_v, act_v, c_v, s0_v, nb_v, tri_v,
          sub_v, obj_v, rel_v, lesub_v, epsub_v, objprow_v, valid_v, sem):
    wid = lax.axis_index("s") * 2 + lax.axis_index("c")

    @pl.when(wid < BSZ)
    def _():
      cp0 = pltpu.async_copy(r0_hbm, r0_v, sem)
      cp1 = pltpu.async_copy(r1_hbm, r1_v, sem)
      pltpu.sync_copy(start_hbm.at[wid], src_v)
      cp0.wait()
      cp1.wait()
      _select_and_gather(src_v, r0_v, r1_v, kbf_hbm, act_v, c_v, s0_v, nb_v,
                         tri_v, sem,
                         sub_v, obj_v, rel_v, lesub_v, epsub_v, valid_v,
                         with_ep=False)
      out_rows(wid, [(sub_v, sub_o), (obj_v, obj_o), (rel_v, rel_o),
                     (lesub_v, lesub_o), (valid_v, valid_o)])

  @functools.partial(
      pl.kernel, mesh=mesh,
      out_type=(rows_i, rows_i, rows_i, rows_f, rows_f, rows_f),
      scratch_types=sel_scratch + [pltpu.VMEM((MAX_ACTIVE,), jnp.int32)],
      compiler_params=cp)
  def sc2(obj_hbm, objp_hbm, r0_hbm, r1_hbm, kbf_hbm,
          sub_o, obj_o, rel_o, lesub_o, epsub_o, valid_o,
          src_v, r0_v, r1_v, act_v, c_v, s0_v, nb_v, tri_v,
          sub_v, obj_v, rel_v, lesub_v, epsub_v, objprow_v, valid_v, sem,
          pobj_v):
    wid = lax.axis_index("s") * 2 + lax.axis_index("c")

    @pl.when(wid < BSZ)
    def _():
      cp0 = pltpu.async_copy(r0_hbm, r0_v, sem)
      cp1 = pltpu.async_copy(r1_hbm, r1_v, sem)
      pltpu.sync_copy(obj_hbm.at[wid], pobj_v)
      pltpu.sync_copy(objp_hbm.at[wid], objprow_v)
      _scatter_ep(src_v, pobj_v, objprow_v)
      cp0.wait()
      cp1.wait()
      _select_and_gather(src_v, r0_v, r1_v, kbf_hbm, act_v, c_v, s0_v, nb_v,
                         tri_v, sem,
                         sub_v, obj_v, rel_v, lesub_v, epsub_v, valid_v,
                         with_ep=True)
      out_rows(wid, [(sub_v, sub_o), (obj_v, obj_o), (rel_v, rel_o),
                     (lesub_v, lesub_o), (epsub_v, epsub_o),
                     (valid_v, valid_o)])

  @functools.partial(
      pl.kernel, mesh=mesh,
      out_type=jax.ShapeDtypeStruct((BSZ, NUM_ENT), jnp.float32),
      scratch_types=[
          pltpu.VMEM((NUM_ENT,), jnp.float32),
          pltpu.VMEM((MAX_ACTIVE,), jnp.int32),
          pltpu.VMEM((MAX_ACTIVE,), jnp.float32),
      ], compiler_params=cp)
  def sc3(obj_hbm, objp_hbm, out_hbm, ep_v, objrow_v, objprow_v):
    wid = lax.axis_index("s") * 2 + lax.axis_index("c")

    @pl.when(wid < BSZ)
    def _():
      pltpu.sync_copy(obj_hbm.at[wid], objrow_v)
      pltpu.sync_copy(objp_hbm.at[wid], objprow_v)
      _scatter_ep(ep_v, objrow_v, objprow_v)

      def norm_body(c, _):
        for u in range(5):
          o = (c * 5 + u) * L
          ep_v[pl.ds(o, L)] = jnp.minimum(ep_v[pl.ds(o, L)], 1.0)
        return 0

      lax.fori_loop(0, ENT_CHUNKS // 5, norm_body, 0)
      pltpu.sync_copy(ep_v, out_hbm.at[wid])

  return sc1, sc2, sc3


def _gates(gi, gh):
  i_r, i_z, i_n = gi[:, :DIM], gi[:, DIM:2 * DIM], gi[:, 2 * DIM:]
  h_r, h_z, h_n = gh[:, :DIM], gh[:, DIM:2 * DIM], gh[:, 2 * DIM:]
  r = jax.nn.sigmoid(i_r + h_r)
  z = jax.nn.sigmoid(i_z + h_z)
  n = jnp.tanh(i_n + r * h_n)
  return z, n


def _tc1_body(rel_ref, lesub_ref, valid_ref, q1h_ref,
              rel_emb_ref, W_ih_ref, b_ih_ref, b_hh_ref,
              sw_ref, sb_ref, clsw_ref, clsb_ref,
              objp_ref, feat_ref):
  rel = rel_ref[0, 0]
  oh = (rel[:, None] ==
        lax.broadcasted_iota(jnp.int32, (MAX_ACTIVE, NUM_REL), 1)
        ).astype(jnp.float32)
  rel_feat = jnp.dot(oh, rel_emb_ref[...],
                     preferred_element_type=jnp.float32)
  gi = jnp.dot(rel_feat, W_ih_ref[...],
               preferred_element_type=jnp.float32) + b_ih_ref[...]
  gh = jnp.broadcast_to(b_hh_ref[...], (MAX_ACTIVE, 3 * DIM))
  z, n = _gates(gi, gh)
  trans = (1.0 - z) * n
  qe = jnp.dot(q1h_ref[0], rel_emb_ref[...],
               preferred_element_type=jnp.float32)
  cq = jnp.tanh(jnp.dot(qe, sw_ref[...],
                        preferred_element_type=jnp.float32) + sb_ref[...])
  logit = jnp.sum(trans * cq * clsw_ref[...], axis=1) + clsb_ref[0, 0]
  prob = jax.nn.sigmoid(logit)
  obj_p = lesub_ref[0, 0] * prob * valid_ref[0, 0]
  objp_ref[0, 0] = obj_p
  feat_ref[0] = trans * obj_p[:, None]


def _tc2_body(rel_ref, sub_ref, lesub_ref, epsub_ref, valid_ref,
              pobj_ref, pfeat_ref, q1h_ref,
              rel_emb_ref, W_ih_ref, W_hh_ref, b_ih_ref, b_hh_ref,
              sw_ref, sb_ref, clsw_ref, clsb_ref,
              objp_ref):
  rel = rel_ref[0, 0]
  oh = (rel[:, None] ==
        lax.broadcasted_iota(jnp.int32, (MAX_ACTIVE, NUM_REL), 1)
        ).astype(jnp.float32)
  rel_feat = jnp.dot(oh, rel_emb_ref[...],
                     preferred_element_type=jnp.float32)
  gi = jnp.dot(rel_feat, W_ih_ref[...],
               preferred_element_type=jnp.float32) + b_ih_ref[...]
  mask = (sub_ref[0, 0][:, None] == pobj_ref[0, 0][None, :]
          ).astype(jnp.float32)
  S = jnp.dot(mask, pfeat_ref[0], preferred_element_type=jnp.float32)
  h = S / (epsub_ref[0, 0][:, None] + 1e-6)
  gh = jnp.dot(h, W_hh_ref[...],
               preferred_element_type=jnp.float32) + b_hh_ref[...]
  z, n = _gates(gi, gh)
  trans = (1.0 - z) * n + z * h
  qe = jnp.dot(q1h_ref[0], rel_emb_ref[...],
               preferred_element_type=jnp.float32)
  cq = jnp.tanh(jnp.dot(qe, sw_ref[...],
                        preferred_element_type=jnp.float32) + sb_ref[...])
  logit = jnp.sum(trans * cq * clsw_ref[...], axis=1) + clsb_ref[0, 0]
  prob = jax.nn.sigmoid(logit)
  objp_ref[0, 0] = lesub_ref[0, 0] * prob * valid_ref[0, 0]


def _i3(x):
  return x.reshape(BSZ, 1, MAX_ACTIVE)


_B3 = lambda: pl.BlockSpec((1, 1, MAX_ACTIVE), lambda i: (i, 0, 0))
_BQ = lambda: pl.BlockSpec((1, 1, NUM_REL), lambda i: (i, 0, 0))
_BW = lambda shape: pl.BlockSpec(shape, lambda i: tuple(0 for _ in shape))


def kernel(start, query, kb_triple, kb_range, rel_emb, step_W, step_b,
           cls_W, cls_b, W_ih, W_hh, b_ih, b_hh):
  f32 = jnp.float32
  kbf = jnp.pad(kb_triple.astype(jnp.int32), ((0, 16), (0, 5))).reshape(-1)
  r0 = kb_range[:, 0].astype(jnp.int32)
  r1 = kb_range[:, 1].astype(jnp.int32)
  q1h = (query[:, None].astype(jnp.int32) ==
         jnp.arange(NUM_REL, dtype=jnp.int32)[None, :]).astype(f32)
  q1h = q1h.reshape(BSZ, 1, NUM_REL)
  b_ih2 = b_ih.reshape(1, 3 * DIM).astype(f32)
  b_hh2 = b_hh.reshape(1, 3 * DIM).astype(f32)
  clsw = cls_W.reshape(1, DIM).astype(f32)
  clsb = cls_b.reshape(1, 1).astype(f32)
  sb = step_b.reshape(2, 1, DIM).astype(f32)
  _sc1, _sc2, _sc3 = _build_sc_kernels()

  sub1, obj1, rel1, lesub1, valid1 = _sc1(start.astype(f32), r0, r1, kbf)

  tc1 = pl.pallas_call(
      _tc1_body,
      grid=(BSZ,),
      in_specs=[_B3(), _B3(), _B3(), _BQ(),
                _BW((NUM_REL, DIM)), _BW((DIM, 3 * DIM)), _BW((1, 3 * DIM)),
                _BW((1, 3 * DIM)), _BW((DIM, DIM)), _BW((1, DIM)),
                _BW((1, DIM)), _BW((1, 1))],
      out_specs=[_B3(), pl.BlockSpec((1, MAX_ACTIVE, DIM), lambda i: (i, 0, 0))],
      out_shape=[jax.ShapeDtypeStruct((BSZ, 1, MAX_ACTIVE), f32),
                 jax.ShapeDtypeStruct((BSZ, MAX_ACTIVE, DIM), f32)],
  )
  objp1, feat1 = tc1(_i3(rel1), _i3(lesub1), _i3(valid1), q1h,
                     rel_emb.astype(f32), W_ih.astype(f32).T, b_ih2, b_hh2,
                     step_W[0].astype(f32), sb[0], clsw, clsb)

  sub2, obj2, rel2, lesub2, epsub2, valid2 = _sc2(
      obj1, objp1.reshape(BSZ, MAX_ACTIVE), r0, r1, kbf)

  tc2 = pl.pallas_call(
      _tc2_body,
      grid=(BSZ,),
      in_specs=[_B3(), _B3(), _B3(), _B3(), _B3(), _B3(),
                pl.BlockSpec((1, MAX_ACTIVE, DIM), lambda i: (i, 0, 0)), _BQ(),
                _BW((NUM_REL, DIM)), _BW((DIM, 3 * DIM)), _BW((DIM, 3 * DIM)),
                _BW((1, 3 * DIM)), _BW((1, 3 * DIM)),
                _BW((DIM, DIM)), _BW((1, DIM)), _BW((1, DIM)), _BW((1, 1))],
      out_specs=[_B3()],
      out_shape=[jax.ShapeDtypeStruct((BSZ, 1, MAX_ACTIVE), f32)],
  )
  (objp2,) = tc2(_i3(rel2), _i3(sub2), _i3(lesub2), _i3(epsub2), _i3(valid2),
                 _i3(obj1), feat1, q1h,
                 rel_emb.astype(f32), W_ih.astype(f32).T, W_hh.astype(f32).T,
                 b_ih2, b_hh2, step_W[1].astype(f32), sb[1], clsw, clsb)

  return _sc3(obj2, objp2.reshape(BSZ, MAX_ACTIVE))
